# Initial kernel scaffold; baseline (speedup 1.0000x reference)
import jax
import jax.numpy as jnp
from jax.experimental import pallas as pl


def kernel(x, edge_index, edge_type, weight, bias, weight_self_loop, ln_gamma, ln_beta):
    raise NotImplementedError("write your pallas kernel here")


# speedup vs baseline: 19.2863x; 19.2863x over previous
import functools

import jax
import jax.numpy as jnp
from jax import lax
from jax.experimental import pallas as pl
from jax.experimental.pallas import tpu as pltpu
from jax.experimental.pallas import tpu_sc as plsc

LN_EPS = 1e-5

NUM_CORES = 2
NUM_SUBCORES = 16
NUM_WORKERS = NUM_CORES * NUM_SUBCORES
LANES = 16


def _h_body(x_ref, w_ref, out_ref):
    out_ref[...] = jnp.dot(
        x_ref[...], w_ref[0], preferred_element_type=jnp.float32)[None]


def _compute_h(x, weight, n_blk):
    n, d_in = x.shape
    r, _, d_out = weight.shape
    return pl.pallas_call(
        _h_body,
        grid=(pl.cdiv(n, n_blk), r),
        in_specs=[
            pl.BlockSpec((n_blk, d_in), lambda nb, rb: (nb, 0)),
            pl.BlockSpec((1, d_in, d_out), lambda nb, rb: (rb, 0, 0)),
        ],
        out_specs=pl.BlockSpec((1, n_blk, d_out), lambda nb, rb: (rb, nb, 0)),
        out_shape=jax.ShapeDtypeStruct((r, n, d_out), jnp.float32),
    )(x, weight)


def _make_sc_aggregate(n_pad, e, d, chunk):
    ep = e // NUM_WORKERS
    n_per_sub = n_pad // NUM_SUBCORES
    num_chunks = ep // chunk
    mesh = plsc.VectorSubcoreMesh(
        core_axis_name="c", subcore_axis_name="s",
        num_cores=NUM_CORES, num_subcores=NUM_SUBCORES,
    )

    @functools.partial(
        pl.kernel,
        mesh=mesh,
        out_type=[
            jax.ShapeDtypeStruct((NUM_CORES * n_pad, d), jnp.float32),
            jax.ShapeDtypeStruct((NUM_WORKERS * n_pad,), jnp.float32),
        ],
        scratch_types=[
            pltpu.VMEM((chunk,), jnp.int32),
            pltpu.VMEM((chunk,), jnp.int32),
            pltpu.VMEM((chunk, d), jnp.float32),
            pltpu.VMEM((n_pad + LANES,), jnp.float32),
            pltpu.VMEM_SHARED((n_pad, d), jnp.float32),
            pltpu.SemaphoreType.DMA,
        ],
    )
    def sc_kernel(h_hbm, gidx_hbm, dst_hbm, zeros2d_hbm, zeros1d_hbm,
                  acc_out, deg_out,
                  idx_v, dst_v, rows_v, deg_v, acc_sh, sem):
        cid = lax.axis_index("c")
        sid = lax.axis_index("s")
        wid = cid * NUM_SUBCORES + sid

        zbase = sid * n_per_sub
        pltpu.sync_copy(zeros2d_hbm.at[pl.ds(zbase, n_per_sub)],
                        acc_sh.at[pl.ds(zbase, n_per_sub)])
        pltpu.sync_copy(zeros1d_hbm, deg_v)
        plsc.subcore_barrier()

        base = wid * ep
        lane = lax.iota(jnp.int32, LANES)

        def body(c, _):
            off = base + c * chunk
            pltpu.sync_copy(gidx_hbm.at[pl.ds(off, chunk)], idx_v)
            pltpu.sync_copy(dst_hbm.at[pl.ds(off, chunk)], dst_v)
            pltpu.async_copy(h_hbm.at[idx_v], rows_v, sem).wait()
            pltpu.sync_copy(rows_v, acc_sh.at[dst_v], add=True)

            def deg_body(j, _):
                dvec = dst_v[pl.ds(j * LANES, LANES)]
                for l in range(LANES):
                    s = dvec[l]
                    b = pl.multiple_of((s // 8) * 8, 8)
                    ev = jnp.where(lane == s - b, 1.0, 0.0)
                    deg_v[pl.ds(b, LANES)] = deg_v[pl.ds(b, LANES)] + ev
                return _

            lax.fori_loop(0, chunk // LANES, deg_body, None)
            return _

        lax.fori_loop(0, num_chunks, body, None)
        plsc.subcore_barrier()

        obase = cid * n_pad + sid * n_per_sub
        pltpu.sync_copy(acc_sh.at[pl.ds(sid * n_per_sub, n_per_sub)],
                        acc_out.at[pl.ds(obase, n_per_sub)])
        pltpu.sync_copy(deg_v.at[pl.ds(0, n_pad)],
                        deg_out.at[pl.ds(wid * n_pad, n_pad)])

    return sc_kernel


def _finalize_body(acc_ref, deg_ref, x_ref, wself_ref, gamma_ref, beta_ref,
                   bias_ref, out_ref):
    s = acc_ref[0] + acc_ref[1]
    deg = jnp.sum(deg_ref[...], axis=0)
    y = s / jnp.maximum(deg, 1.0)[:, None]
    mu = jnp.mean(y, axis=-1, keepdims=True)
    d = y - mu
    var = jnp.mean(d * d, axis=-1, keepdims=True)
    yn = d * lax.rsqrt(var + LN_EPS)
    out = yn * gamma_ref[...] + beta_ref[...] + bias_ref[...]
    out = out + jnp.dot(x_ref[...], wself_ref[...], preferred_element_type=jnp.float32)
    out_ref[...] = out


def _finalize(acc, deg, x, wself, gamma, beta, bias, n_blk):
    n, d_in = x.shape
    d_out = wself.shape[1]
    return pl.pallas_call(
        _finalize_body,
        grid=(pl.cdiv(n, n_blk),),
        in_specs=[
            pl.BlockSpec((2, n_blk, d_out), lambda nb: (0, nb, 0)),
            pl.BlockSpec((NUM_WORKERS, n_blk), lambda nb: (0, nb)),
            pl.BlockSpec((n_blk, d_in), lambda nb: (nb, 0)),
            pl.BlockSpec((d_in, d_out), lambda nb: (0, 0)),
            pl.BlockSpec((1, d_out), lambda nb: (0, 0)),
            pl.BlockSpec((1, d_out), lambda nb: (0, 0)),
            pl.BlockSpec((1, d_out), lambda nb: (0, 0)),
        ],
        out_specs=pl.BlockSpec((n_blk, d_out), lambda nb: (nb, 0)),
        out_shape=jax.ShapeDtypeStruct((n, d_out), jnp.float32),
    )(acc, deg, x, wself, gamma, beta, bias)


def kernel(x, edge_index, edge_type, weight, bias, weight_self_loop, ln_gamma, ln_beta):
    n, d_in = x.shape
    e = edge_index.shape[1]
    r = weight.shape[0]
    d_out = weight.shape[2]

    src = edge_index[0]
    dst = edge_index[1]
    gidx = edge_type * n + src

    h = _compute_h(x, weight, n_blk=512)
    h_flat = h.reshape(r * n, d_out)

    n_pad = 10240
    zeros2d = jnp.zeros((n_pad, d_out), dtype=jnp.float32)
    zeros1d = jnp.zeros((n_pad + LANES,), dtype=jnp.float32)
    sc = _make_sc_aggregate(n_pad, e, d_out, chunk=80)
    acc_flat, deg_flat = sc(h_flat, gidx, dst, zeros2d, zeros1d)
    acc = acc_flat.reshape(NUM_CORES, n_pad, d_out)
    deg = deg_flat.reshape(NUM_WORKERS, n_pad)

    out = _finalize(
        acc, deg, x, weight_self_loop,
        ln_gamma.reshape(1, -1), ln_beta.reshape(1, -1), bias.reshape(1, -1),
        n_blk=512,
    )
    return out


# speedup vs baseline: 31.8476x; 1.6513x over previous
import functools

import jax
import jax.numpy as jnp
from jax import lax
from jax.experimental import pallas as pl
from jax.experimental.pallas import tpu as pltpu
from jax.experimental.pallas import tpu_sc as plsc

LN_EPS = 1e-5

NUM_CORES = 2
NUM_SUBCORES = 16
NUM_WORKERS = NUM_CORES * NUM_SUBCORES
LANES = 16


def _h_body(x_ref, w_ref, out_ref):
    out_ref[...] = jnp.dot(
        x_ref[...], w_ref[0], preferred_element_type=jnp.float32)[None]


def _compute_h(x, weight, n_blk):
    n, d_in = x.shape
    r, _, d_out = weight.shape
    return pl.pallas_call(
        _h_body,
        grid=(pl.cdiv(n, n_blk), r),
        in_specs=[
            pl.BlockSpec((n_blk, d_in), lambda nb, rb: (nb, 0)),
            pl.BlockSpec((1, d_in, d_out), lambda nb, rb: (rb, 0, 0)),
        ],
        out_specs=pl.BlockSpec((1, n_blk, d_out), lambda nb, rb: (rb, nb, 0)),
        out_shape=jax.ShapeDtypeStruct((r, n, d_out), jnp.float32),
    )(x, weight)


def _make_sc_aggregate(n_pad, e, d, chunk):
    ep = e // NUM_WORKERS
    n_per_sub = n_pad // NUM_SUBCORES
    num_chunks = ep // chunk
    mesh = plsc.VectorSubcoreMesh(
        core_axis_name="c", subcore_axis_name="s",
        num_cores=NUM_CORES, num_subcores=NUM_SUBCORES,
    )

    assert num_chunks % 2 == 1

    @functools.partial(
        pl.kernel,
        mesh=mesh,
        out_type=[
            jax.ShapeDtypeStruct((NUM_CORES * n_pad, d), jnp.float32),
            jax.ShapeDtypeStruct((NUM_WORKERS * n_pad,), jnp.float32),
        ],
        scratch_types=[
            pltpu.VMEM((ep,), jnp.int32),
            pltpu.VMEM((chunk,), jnp.int32),
            pltpu.VMEM((chunk,), jnp.int32),
            pltpu.VMEM((chunk, d), jnp.float32),
            pltpu.VMEM((chunk, d), jnp.float32),
            pltpu.VMEM((n_pad + LANES,), jnp.float32),
            pltpu.VMEM_SHARED((n_pad, d), jnp.float32),
            pltpu.SemaphoreType.DMA,
            pltpu.SemaphoreType.DMA,
            pltpu.SemaphoreType.DMA,
            pltpu.SemaphoreType.DMA,
        ],
    )
    def sc_kernel(h_hbm, gidx_hbm, dst_hbm, zeros2d_hbm, zeros1d_hbm,
                  acc_out, deg_out,
                  idx_all, dst_a, dst_b, rows_a, rows_b,
                  deg_v, acc_sh, sem_a, sem_b, sdm_a, sdm_b):
        cid = lax.axis_index("c")
        sid = lax.axis_index("s")
        wid = cid * NUM_SUBCORES + sid
        lane = lax.iota(jnp.int32, LANES)

        zbase = sid * n_per_sub
        pltpu.sync_copy(zeros2d_hbm.at[pl.ds(zbase, n_per_sub)],
                        acc_sh.at[pl.ds(zbase, n_per_sub)])
        pltpu.sync_copy(zeros1d_hbm, deg_v)
        base = wid * ep
        pltpu.sync_copy(gidx_hbm.at[pl.ds(base, ep)], idx_all)
        plsc.subcore_barrier()

        def dload(c, dst_buf, sem):
            pltpu.async_copy(
                dst_hbm.at[pl.ds(base + c * chunk, chunk)], dst_buf, sem)

        def dwait(dst_buf, sem):
            pltpu.make_async_copy(
                dst_hbm.at[pl.ds(0, chunk)], dst_buf, sem).wait()

        def deg_update(dst_buf):
            for j in range(chunk // LANES):
                dvec = dst_buf[pl.ds(j * LANES, LANES)]
                for l in range(LANES):
                    s = dvec[l]
                    b = pl.multiple_of((s // 8) * 8, 8)
                    ev = jnp.where(lane == s - b, 1.0, 0.0)
                    deg_v[pl.ds(b, LANES)] = deg_v[pl.ds(b, LANES)] + ev

        def gather(c, rows_buf, sem):
            pltpu.async_copy(
                h_hbm.at[idx_all.at[pl.ds(c * chunk, chunk)]], rows_buf, sem)

        def gwait(rows_buf, sem):
            pltpu.make_async_copy(h_hbm.at[pl.ds(0, chunk)], rows_buf, sem).wait()

        def scatter(rows_buf, dst_buf):
            pltpu.sync_copy(rows_buf, acc_sh.at[dst_buf], add=True)

        dload(0, dst_a, sdm_a)
        gather(0, rows_a, sem_a)

        def pair_body(p, _):
            c0 = 2 * p
            dload(c0 + 1, dst_b, sdm_b)
            gwait(rows_a, sem_a)
            gather(c0 + 1, rows_b, sem_b)
            dwait(dst_a, sdm_a)
            deg_update(dst_a)
            scatter(rows_a, dst_a)
            dload(c0 + 2, dst_a, sdm_a)
            gwait(rows_b, sem_b)
            gather(c0 + 2, rows_a, sem_a)
            dwait(dst_b, sdm_b)
            deg_update(dst_b)
            scatter(rows_b, dst_b)
            return _

        lax.fori_loop(0, (num_chunks - 1) // 2, pair_body, None)

        gwait(rows_a, sem_a)
        dwait(dst_a, sdm_a)
        deg_update(dst_a)
        scatter(rows_a, dst_a)
        plsc.subcore_barrier()

        obase = cid * n_pad + sid * n_per_sub
        pltpu.sync_copy(acc_sh.at[pl.ds(sid * n_per_sub, n_per_sub)],
                        acc_out.at[pl.ds(obase, n_per_sub)])
        pltpu.sync_copy(deg_v.at[pl.ds(0, n_pad)],
                        deg_out.at[pl.ds(wid * n_pad, n_pad)])

    return sc_kernel


def _finalize_body(acc_ref, deg_ref, x_ref, wself_ref, gamma_ref, beta_ref,
                   bias_ref, out_ref):
    s = acc_ref[0] + acc_ref[1]
    deg = jnp.sum(deg_ref[...], axis=0)
    y = s / jnp.maximum(deg, 1.0)[:, None]
    mu = jnp.mean(y, axis=-1, keepdims=True)
    d = y - mu
    var = jnp.mean(d * d, axis=-1, keepdims=True)
    yn = d * lax.rsqrt(var + LN_EPS)
    out = yn * gamma_ref[...] + beta_ref[...] + bias_ref[...]
    out = out + jnp.dot(x_ref[...], wself_ref[...], preferred_element_type=jnp.float32)
    out_ref[...] = out


def _finalize(acc, deg, x, wself, gamma, beta, bias, n_blk):
    n, d_in = x.shape
    d_out = wself.shape[1]
    return pl.pallas_call(
        _finalize_body,
        grid=(pl.cdiv(n, n_blk),),
        in_specs=[
            pl.BlockSpec((2, n_blk, d_out), lambda nb: (0, nb, 0)),
            pl.BlockSpec((NUM_WORKERS, n_blk), lambda nb: (0, nb)),
            pl.BlockSpec((n_blk, d_in), lambda nb: (nb, 0)),
            pl.BlockSpec((d_in, d_out), lambda nb: (0, 0)),
            pl.BlockSpec((1, d_out), lambda nb: (0, 0)),
            pl.BlockSpec((1, d_out), lambda nb: (0, 0)),
            pl.BlockSpec((1, d_out), lambda nb: (0, 0)),
        ],
        out_specs=pl.BlockSpec((n_blk, d_out), lambda nb: (nb, 0)),
        out_shape=jax.ShapeDtypeStruct((n, d_out), jnp.float32),
    )(acc, deg, x, wself, gamma, beta, bias)


def kernel(x, edge_index, edge_type, weight, bias, weight_self_loop, ln_gamma, ln_beta):
    n, d_in = x.shape
    e = edge_index.shape[1]
    r = weight.shape[0]
    d_out = weight.shape[2]

    src = edge_index[0]
    dst = edge_index[1]
    gidx = edge_type * n + src

    h = _compute_h(x, weight, n_blk=512)
    h_flat = h.reshape(r * n, d_out)

    n_pad = 10240
    zeros2d = jnp.zeros((n_pad, d_out), dtype=jnp.float32)
    zeros1d = jnp.zeros((n_pad + LANES,), dtype=jnp.float32)
    sc = _make_sc_aggregate(n_pad, e, d_out, chunk=80)
    acc_flat, deg_flat = sc(h_flat, gidx, dst, zeros2d, zeros1d)
    acc = acc_flat.reshape(NUM_CORES, n_pad, d_out)
    deg = deg_flat.reshape(NUM_WORKERS, n_pad)

    out = _finalize(
        acc, deg, x, weight_self_loop,
        ln_gamma.reshape(1, -1), ln_beta.reshape(1, -1), bias.reshape(1, -1),
        n_blk=512,
    )
    return out


# speedup vs baseline: 47.1632x; 1.4809x over previous
import functools

import jax
import jax.numpy as jnp
from jax import lax
from jax.experimental import pallas as pl
from jax.experimental.pallas import tpu as pltpu
from jax.experimental.pallas import tpu_sc as plsc

LN_EPS = 1e-5

NUM_CORES = 2
NUM_SUBCORES = 16
NUM_WORKERS = NUM_CORES * NUM_SUBCORES
LANES = 16


def _h_body(x_ref, w_ref, ei_ref, et_ref, h_ref, gidx_ref, dst_ref):
    n = x_ref.shape[0]
    h_ref[...] = jnp.dot(
        x_ref[...], w_ref[0], preferred_element_type=jnp.float32)[None]

    @pl.when(pl.program_id(0) == 0)
    def _():
        gidx_ref[...] = et_ref[...] * n + ei_ref[0, :]
        dst_ref[...] = ei_ref[1, :]


def _compute_h(x, weight, edge_index, edge_type):
    n, d_in = x.shape
    r, _, d_out = weight.shape
    e = edge_index.shape[1]
    return pl.pallas_call(
        _h_body,
        grid=(r,),
        in_specs=[
            pl.BlockSpec((n, d_in), lambda rb: (0, 0)),
            pl.BlockSpec((1, d_in, d_out), lambda rb: (rb, 0, 0)),
            pl.BlockSpec((2, e), lambda rb: (0, 0)),
            pl.BlockSpec((e,), lambda rb: (0,)),
        ],
        out_specs=[
            pl.BlockSpec((1, n, d_out), lambda rb: (rb, 0, 0)),
            pl.BlockSpec((e,), lambda rb: (0,)),
            pl.BlockSpec((e,), lambda rb: (0,)),
        ],
        out_shape=[
            jax.ShapeDtypeStruct((r, n, d_out), jnp.float32),
            jax.ShapeDtypeStruct((e,), jnp.int32),
            jax.ShapeDtypeStruct((e,), jnp.int32),
        ],
    )(x, weight, edge_index, edge_type)


def _make_sc_aggregate(n_pad, e, d, r, chunk):
    ep = e // NUM_WORKERS
    n_per_sub = n_pad // NUM_SUBCORES
    num_chunks = ep // chunk
    mesh = plsc.VectorSubcoreMesh(
        core_axis_name="c", subcore_axis_name="s",
        num_cores=NUM_CORES, num_subcores=NUM_SUBCORES,
    )

    assert num_chunks % 2 == 1

    @functools.partial(
        pl.kernel,
        mesh=mesh,
        out_type=[
            jax.ShapeDtypeStruct((NUM_CORES * n_pad, d), jnp.float32),
            jax.ShapeDtypeStruct((NUM_WORKERS * n_pad,), jnp.float32),
        ],
        scratch_types=[
            pltpu.VMEM((ep,), jnp.int32),
            pltpu.VMEM((chunk,), jnp.int32),
            pltpu.VMEM((chunk,), jnp.int32),
            pltpu.VMEM((chunk, d), jnp.float32),
            pltpu.VMEM((chunk, d), jnp.float32),
            pltpu.VMEM((n_pad + LANES,), jnp.float32),
            pltpu.VMEM_SHARED((n_pad, d), jnp.float32),
            pltpu.SemaphoreType.DMA,
            pltpu.SemaphoreType.DMA,
            pltpu.SemaphoreType.DMA,
            pltpu.SemaphoreType.DMA,
        ],
    )
    def sc_kernel(h_hbm, gidx_hbm, dst_hbm, zeros2d_hbm, zeros1d_hbm,
                  acc_out, deg_out,
                  idx_all, dst_a, dst_b, rows_a, rows_b,
                  deg_v, acc_sh, sem_a, sem_b, sdm_a, sdm_b):
        cid = lax.axis_index("c")
        sid = lax.axis_index("s")
        wid = cid * NUM_SUBCORES + sid
        lane = lax.iota(jnp.int32, LANES)

        pltpu.sync_copy(zeros2d_hbm,
                        acc_sh.at[pl.ds(sid * n_per_sub, n_per_sub)])
        pltpu.sync_copy(zeros1d_hbm, deg_v)
        base = wid * ep
        pltpu.sync_copy(gidx_hbm.at[pl.ds(base, ep)], idx_all)
        plsc.subcore_barrier()

        def dload(c, dst_buf, sem):
            pltpu.async_copy(
                dst_hbm.at[pl.ds(base + c * chunk, chunk)], dst_buf, sem)

        def dwait(dst_buf, sem):
            pltpu.make_async_copy(
                dst_hbm.at[pl.ds(0, chunk)], dst_buf, sem).wait()

        def deg_update(dst_buf):
            for j in range(chunk // LANES):
                dvec = dst_buf[pl.ds(j * LANES, LANES)]
                for l in range(LANES):
                    s = dvec[l]
                    b = pl.multiple_of((s // 8) * 8, 8)
                    ev = jnp.where(lane == s - b, 1.0, 0.0)
                    deg_v[pl.ds(b, LANES)] = deg_v[pl.ds(b, LANES)] + ev

        def gather(c, rows_buf, sem):
            pltpu.async_copy(
                h_hbm.at[idx_all.at[pl.ds(c * chunk, chunk)]], rows_buf, sem)

        def gwait(rows_buf, sem):
            pltpu.make_async_copy(h_hbm.at[pl.ds(0, chunk)], rows_buf, sem).wait()

        def scatter(rows_buf, dst_buf):
            pltpu.sync_copy(rows_buf, acc_sh.at[dst_buf], add=True)

        dload(0, dst_a, sdm_a)
        gather(0, rows_a, sem_a)

        def pair_body(p, _):
            c0 = 2 * p
            dload(c0 + 1, dst_b, sdm_b)
            gwait(rows_a, sem_a)
            gather(c0 + 1, rows_b, sem_b)
            dwait(dst_a, sdm_a)
            deg_update(dst_a)
            scatter(rows_a, dst_a)
            dload(c0 + 2, dst_a, sdm_a)
            gwait(rows_b, sem_b)
            gather(c0 + 2, rows_a, sem_a)
            dwait(dst_b, sdm_b)
            deg_update(dst_b)
            scatter(rows_b, dst_b)
            return _

        lax.fori_loop(0, (num_chunks - 1) // 2, pair_body, None)

        gwait(rows_a, sem_a)
        dwait(dst_a, sdm_a)
        deg_update(dst_a)
        scatter(rows_a, dst_a)
        plsc.subcore_barrier()

        obase = cid * n_pad + sid * n_per_sub
        pltpu.sync_copy(acc_sh.at[pl.ds(sid * n_per_sub, n_per_sub)],
                        acc_out.at[pl.ds(obase, n_per_sub)])
        pltpu.sync_copy(deg_v.at[pl.ds(0, n_pad)],
                        deg_out.at[pl.ds(wid * n_pad, n_pad)])

    return sc_kernel


def _finalize_body(acc_ref, deg_ref, x_ref, wself_ref, gamma_ref, beta_ref,
                   bias_ref, out_ref):
    s = acc_ref[0] + acc_ref[1]
    deg = jnp.sum(deg_ref[...], axis=0)
    y = s / jnp.maximum(deg, 1.0)[:, None]
    mu = jnp.mean(y, axis=-1, keepdims=True)
    d = y - mu
    var = jnp.mean(d * d, axis=-1, keepdims=True)
    yn = d * lax.rsqrt(var + LN_EPS)
    out = yn * gamma_ref[...] + beta_ref[...] + bias_ref[...]
    out = out + jnp.dot(x_ref[...], wself_ref[...], preferred_element_type=jnp.float32)
    out_ref[...] = out


def _finalize(acc, deg, x, wself, gamma, beta, bias, n_blk):
    n, d_in = x.shape
    d_out = wself.shape[1]
    return pl.pallas_call(
        _finalize_body,
        grid=(pl.cdiv(n, n_blk),),
        in_specs=[
            pl.BlockSpec((2, n_blk, d_out), lambda nb: (0, nb, 0)),
            pl.BlockSpec((NUM_WORKERS, n_blk), lambda nb: (0, nb)),
            pl.BlockSpec((n_blk, d_in), lambda nb: (nb, 0)),
            pl.BlockSpec((d_in, d_out), lambda nb: (0, 0)),
            pl.BlockSpec((1, d_out), lambda nb: (0, 0)),
            pl.BlockSpec((1, d_out), lambda nb: (0, 0)),
            pl.BlockSpec((1, d_out), lambda nb: (0, 0)),
        ],
        out_specs=pl.BlockSpec((n_blk, d_out), lambda nb: (nb, 0)),
        out_shape=jax.ShapeDtypeStruct((n, d_out), jnp.float32),
    )(acc, deg, x, wself, gamma, beta, bias)


def kernel(x, edge_index, edge_type, weight, bias, weight_self_loop, ln_gamma, ln_beta):
    n, d_in = x.shape
    e = edge_index.shape[1]
    r = weight.shape[0]
    d_out = weight.shape[2]

    h3, gidx, dstv = _compute_h(x, weight, edge_index, edge_type)
    h_flat = h3.reshape(r * n, d_out)

    n_pad = 10240
    zeros2d = jnp.zeros((n_pad // NUM_SUBCORES, d_out), dtype=jnp.float32)
    zeros1d = jnp.zeros((n_pad + LANES,), dtype=jnp.float32)
    sc = _make_sc_aggregate(n_pad, e, d_out, r, chunk=80)
    acc_flat, deg_flat = sc(h_flat, gidx, dstv, zeros2d, zeros1d)
    acc = acc_flat.reshape(NUM_CORES, n_pad, d_out)
    deg = deg_flat.reshape(NUM_WORKERS, n_pad)

    out = _finalize(
        acc, deg, x, weight_self_loop,
        ln_gamma.reshape(1, -1), ln_beta.reshape(1, -1), bias.reshape(1, -1),
        n_blk=1024,
    )
    return out
